# baseline (speedup 1.0000x reference)
import functools

import jax
import jax.numpy as jnp
from jax import lax
from jax.experimental import pallas as pl
from jax.experimental.pallas import tpu as pltpu
from jax.experimental.pallas import tpu_sc as plsc

N = 50000
E = 800000
H = 8
D = 32
HD = H * D
G = 64

NC, NS, LN = 2, 16, 16

BN = 2048
NB = 25
NP = BN * NB
EP = 802816
ER = EP // 128
RNG = 4608
NPASS = 6
DUMP = RNG
NT = 50240
NCOV = 50176
PADDST = 50176
QN = 256
FB = 64

_f32 = jnp.float32
_i32 = jnp.int32


def _tca_body(x_ref, w_ref, al_ref, ar_ref, feat_ref, el_ref, er_ref, cap_ref,
              acc_ref):
    i = pl.program_id(0)
    x = x_ref[...]
    feat = jnp.dot(x, w_ref[...], preferred_element_type=_f32)
    feat_ref[...] = feat
    el = jnp.dot(feat, al_ref[...], preferred_element_type=_f32)
    er = jnp.dot(feat, ar_ref[...], preferred_element_type=_f32)
    z = jnp.zeros_like(el)
    el_ref[...] = jnp.concatenate([el, z], axis=1)
    er_ref[...] = jnp.concatenate([er, z], axis=1)
    bm = jnp.concatenate([jnp.max(el, axis=0, keepdims=True),
                          jnp.max(er, axis=0, keepdims=True)], axis=1)

    @pl.when(i == 0)
    def _():
        acc_ref[...] = jnp.full((1, 16), -jnp.inf, _f32)
        cap_ref[...] = jnp.zeros((8, 128), _f32)

    acc = jnp.maximum(acc_ref[...], bm)
    acc_ref[...] = acc

    @pl.when(i == NB - 1)
    def _():
        csum = acc[:, 0:8] + acc[:, 8:16]
        cap = jnp.maximum(csum, 0.2 * csum)
        big = jnp.full((1, 8), 1e30, _f32)
        cap_ref[0:1, 0:16] = jnp.concatenate([cap, big], axis=1)


def _tca(x_pad, w, al, ar):
    return pl.pallas_call(
        _tca_body,
        grid=(NB,),
        in_specs=[
            pl.BlockSpec((BN, 8), lambda i: (i, 0)),
            pl.BlockSpec((8, HD), lambda i: (0, 0)),
            pl.BlockSpec((HD, 8), lambda i: (0, 0)),
            pl.BlockSpec((HD, 8), lambda i: (0, 0)),
        ],
        out_specs=[
            pl.BlockSpec((BN, HD), lambda i: (i, 0)),
            pl.BlockSpec((BN, 16), lambda i: (i, 0)),
            pl.BlockSpec((BN, 16), lambda i: (i, 0)),
            pl.BlockSpec((8, 128), lambda i: (0, 0)),
        ],
        out_shape=[
            jax.ShapeDtypeStruct((NP, HD), _f32),
            jax.ShapeDtypeStruct((NP, 16), _f32),
            jax.ShapeDtypeStruct((NP, 16), _f32),
            jax.ShapeDtypeStruct((8, 128), _f32),
        ],
        scratch_shapes=[pltpu.VMEM((1, 16), _f32)],
    )(x_pad, w, al, ar)


def _b1_body(src_hbm, dst_hbm, el_hbm, er_hbm, cap_hbm, ee_hbm, dp_hbm,
             srcb, dstb, elrows, errows, eerows, cvec, way,
             sem0, sem1, semw, sems, denom_sh):
    c = lax.axis_index("c")
    s = lax.axis_index("s")

    def zrow(i, carry):
        way[i, :] = jnp.zeros((16,), _f32)
        return carry

    lax.fori_loop(0, 785, zrow, 0)
    for t in range(4):
        pltpu.sync_copy(way, denom_sh.at[pl.ds(s * 3140 + t * 785, 785)])
    pltpu.sync_copy(cap_hbm.at[0, pl.ds(0, 16)], cvec)
    plsc.subcore_barrier()

    base_row = (c * NS + s) * 196

    def chunk(g, carry):
        ph = lax.rem(g, 2)
        row = base_row + g * 4

        @pl.when(g >= 2)
        def _():
            pltpu.make_async_copy(
                eerows.at[pl.ds(ph * 512, 512)],
                ee_hbm.at[pl.ds(row * 128, 512)], semw).wait()
            for j in range(4):
                pltpu.make_async_copy(
                    eerows.at[pl.ds(ph * 512 + j * 128, 128)],
                    denom_sh.at[dstb.at[ph * 4 + j]], sems).wait()

        pltpu.sync_copy(src_hbm.at[pl.ds(row, 4)],
                        srcb.at[pl.ds(ph * 4, 4)])
        pltpu.sync_copy(dst_hbm.at[pl.ds(row, 4)],
                        dstb.at[pl.ds(ph * 4, 4)])
        cps = []
        for j in range(4):
            cps.append(pltpu.async_copy(
                el_hbm.at[srcb.at[ph * 4 + j]],
                elrows.at[pl.ds(j * 128, 128)], sem0))
            cps.append(pltpu.async_copy(
                er_hbm.at[dstb.at[ph * 4 + j]],
                errows.at[pl.ds(j * 128, 128)], sem1))
        for cp in cps:
            cp.wait()
        cv = cvec[...]

        def edge(i, carry2):
            v = elrows[i, :] + errows[i, :]
            v = jnp.maximum(v, 0.2 * v)
            eerows[ph * 512 + i, :] = jnp.exp(v - cv)
            return carry2

        lax.fori_loop(0, 512, edge, 0, unroll=4)
        pltpu.async_copy(eerows.at[pl.ds(ph * 512, 512)],
                         ee_hbm.at[pl.ds(row * 128, 512)], semw)
        for j in range(4):
            pltpu.async_copy(eerows.at[pl.ds(ph * 512 + j * 128, 128)],
                             denom_sh.at[dstb.at[ph * 4 + j]], sems,
                             add=True)
        return carry

    lax.fori_loop(0, 49, chunk, 0)
    for _ in range(2):
        pltpu.make_async_copy(eerows.at[pl.ds(0, 512)],
                              ee_hbm.at[pl.ds(0, 512)], semw).wait()
        for j in range(4):
            pltpu.make_async_copy(eerows.at[pl.ds(j * 128, 128)],
                                  denom_sh.at[dstb.at[j]], sems).wait()
    plsc.subcore_barrier()
    for t in range(4):
        pltpu.sync_copy(denom_sh.at[pl.ds(s * 3140 + t * 785, 785)], way)
        pltpu.sync_copy(way, dp_hbm.at[c, pl.ds(s * 3140 + t * 785, 785)])


def _b1(src2, dst2, el_t, er_t, cap):
    mesh = plsc.VectorSubcoreMesh(core_axis_name="c", subcore_axis_name="s",
                                  num_cores=NC, num_subcores=NS)
    f = pl.kernel(
        _b1_body,
        compiler_params=pltpu.CompilerParams(use_tc_tiling_on_sc=False, needs_layout_passes=False),
        out_type=(jax.ShapeDtypeStruct((EP, 16), _f32),
                  jax.ShapeDtypeStruct((NC, NT, 16), _f32)),
        mesh=mesh,
        scratch_types=[
            pltpu.VMEM((8, 128), _i32),
            pltpu.VMEM((8, 128), _i32),
            pltpu.VMEM((512, 16), _f32),
            pltpu.VMEM((512, 16), _f32),
            pltpu.VMEM((1024, 16), _f32),
            pltpu.VMEM((16,), _f32),
            pltpu.VMEM((785, 16), _f32),
            pltpu.SemaphoreType.DMA,
            pltpu.SemaphoreType.DMA,
            pltpu.SemaphoreType.DMA,
            pltpu.SemaphoreType.DMA,
            pltpu.VMEM_SHARED((NT, 16), _f32),
        ],
    )
    return f(src2, dst2, el_t, er_t, cap)


def _a2_body(dp_ref, rd_ref):
    d = dp_ref[0] + dp_ref[1]
    rd_ref[...] = 1.0 / jnp.maximum(d, 1e-30)


def _a2(dp):
    return pl.pallas_call(
        _a2_body,
        grid=(8,),
        in_specs=[pl.BlockSpec((2, 6280, 16), lambda i: (0, i, 0))],
        out_specs=pl.BlockSpec((6280, 16), lambda i: (i, 0)),
        out_shape=jax.ShapeDtypeStruct((NT, 16), _f32),
    )(dp)


def _b2_body(src_hbm, dst_hbm, feat_hbm, ee_hbm, rd_hbm, fn_hbm, pp_hbm,
             srcba, dstba, srcbb, dstbb, qsrc, qdl, qpos, idxs, idxp, idxd2,
             idxr, fbuf, eebuf, rdbuf, rbuf, fidx32, pbuf,
             sem0, sem1, sem2, sem3, semas, semad, sembs, sembd, rst_sh,
             pooled_sh):
    c = lax.axis_index("c")
    s = lax.axis_index("s")
    iota16 = lax.broadcasted_iota(_i32, (16,), 0)
    zero16 = jnp.zeros((16,), _f32)

    def zrow(i, carry):
        for k in range(16):
            rbuf[i, pl.ds(k * 16, 16)] = zero16
        return carry

    lax.fori_loop(0, 32, zrow, 0)
    pltpu.sync_copy(rbuf.at[pl.ds(0, 4)], pooled_sh.at[pl.ds(s * 4, 4)])

    dump16 = jnp.full((16,), DUMP, _i32)
    z16i = jnp.zeros((16,), _i32)

    def queue_reset():
        for k in range(QN // 16):
            qsrc[pl.ds(k * 16, 16)] = z16i
            qdl[pl.ds(k * 16, 16)] = dump16
            qpos[pl.ds(k * 16, 16)] = z16i

    def issue(np, lo):
        for k in range(FB // 16):
            sv = qsrc[pl.ds(k * 16, 16)]
            pv = qpos[pl.ds(k * 16, 16)]
            dv = qdl[pl.ds(k * 16, 16)]
            idxs[pl.ds(np * FB + k * 16, 16)] = sv
            idxp[pl.ds(np * FB + k * 16, 16)] = pv
            idxd2[np, pl.ds(k * 16, 16)] = dv
            idxr[pl.ds(np * FB + k * 16, 16)] = jnp.minimum(dv + lo, NT - 1)
        pltpu.async_copy(feat_hbm.at[idxs.at[pl.ds(np * FB, FB)]],
                         fbuf.at[pl.ds(np * FB, FB)], sem0)
        pltpu.async_copy(ee_hbm.at[idxp.at[pl.ds(np * FB, FB)]],
                         eebuf.at[pl.ds(np * FB, FB)], sem1)
        pltpu.async_copy(rd_hbm.at[idxr.at[pl.ds(np * FB, FB)]],
                         rdbuf.at[pl.ds(np * FB, FB)], sem2)
        for k in range((QN - FB) // 16):
            qsrc[pl.ds(k * 16, 16)] = qsrc[pl.ds(FB + k * 16, 16)]
            qdl[pl.ds(k * 16, 16)] = qdl[pl.ds(FB + k * 16, 16)]
            qpos[pl.ds(k * 16, 16)] = qpos[pl.ds(FB + k * 16, 16)]
        for k in range(FB // 16):
            qsrc[pl.ds(QN - FB + k * 16, 16)] = z16i
            qdl[pl.ds(QN - FB + k * 16, 16)] = dump16
            qpos[pl.ds(QN - FB + k * 16, 16)] = z16i

    def complete(ip, spin):
        @pl.when(spin == 1)
        def _():
            pltpu.make_async_copy(fbuf.at[pl.ds((1 - ip) * FB, FB)],
                                  rst_sh.at[idxd2.at[1 - ip]], sem3).wait()

        pltpu.make_async_copy(feat_hbm.at[idxs.at[pl.ds(ip * FB, FB)]],
                              fbuf.at[pl.ds(ip * FB, FB)], sem0).wait()
        pltpu.make_async_copy(ee_hbm.at[idxp.at[pl.ds(ip * FB, FB)]],
                              eebuf.at[pl.ds(ip * FB, FB)], sem1).wait()
        pltpu.make_async_copy(rd_hbm.at[idxr.at[pl.ds(ip * FB, FB)]],
                              rdbuf.at[pl.ds(ip * FB, FB)], sem2).wait()

        def mrow(i, carry):
            r = ip * FB + i
            iv = jnp.zeros((16,), _i32) + r
            for h in range(8):
                hv = jnp.full((16,), h, _i32)
                asp = (plsc.load_gather(eebuf, [iv, hv])
                       * plsc.load_gather(rdbuf, [iv, hv]))
                for cc in (2 * h, 2 * h + 1):
                    fbuf[r, pl.ds(cc * 16, 16)] = (
                        fbuf[r, pl.ds(cc * 16, 16)] * asp)
            return carry

        lax.fori_loop(0, FB, mrow, 0, unroll=2)
        pltpu.async_copy(fbuf.at[pl.ds(ip * FB, FB)],
                         rst_sh.at[idxd2.at[ip]], sem3, add=True)

    def maybe_flush(state, lo, thresh):
        q, pend, ip, sp = state
        hit = q >= thresh

        @pl.when(hit & (pend == 1))
        def _():
            complete(ip, sp)

        np = jnp.where(pend == 1, 1 - ip, 0)

        @pl.when(hit)
        def _():
            issue(np, lo)

        q = jnp.where(hit, q - FB, q)
        sp = jnp.where(hit & (pend == 1), 1, sp)
        pend = jnp.where(hit, 1, pend)
        ip = jnp.where(hit, np, ip)
        return (q, pend, ip, sp)

    def one_pass(p, carry):
        lo = (c * NPASS + p) * RNG
        lax.fori_loop(0, 32, zrow, 0)
        r0 = s * 289
        for t in range(9):
            pltpu.sync_copy(rbuf, rst_sh.at[pl.ds(r0 + t * 32, 32)])
        pltpu.sync_copy(rbuf.at[pl.ds(0, 1)], rst_sh.at[pl.ds(r0 + 288, 1)])
        queue_reset()
        plsc.subcore_barrier()

        hi = jnp.minimum(lo + RNG, NCOV)

        def process4(sb, db, row, state, lo):
            for j in range(4):
                q = state[0]
                for gg in range(8):
                    dv = db[j, pl.ds(gg * 16, 16)]
                    sv = sb[j, pl.ds(gg * 16, 16)]
                    mask = (dv >= lo) & (dv < hi)
                    m01 = jnp.where(mask, 1, 0).astype(_i32)
                    csum = plsc.cumsum(m01)
                    tgt = q + csum - 1
                    plsc.store_scatter(qsrc, [tgt], sv, mask=mask)
                    plsc.store_scatter(qdl, [tgt], dv - lo, mask=mask)
                    pos = (row + j) * 128 + gg * 16 + iota16
                    plsc.store_scatter(qpos, [tgt], pos, mask=mask)
                    q = q + jnp.sum(m01)
                state = (q,) + state[1:]
                state = maybe_flush(state, lo, FB)
            return state

        base = s * 392
        pltpu.async_copy(src_hbm.at[pl.ds(base, 4)], srcba, semas)
        pltpu.async_copy(dst_hbm.at[pl.ds(base, 4)], dstba, semad)
        pltpu.async_copy(src_hbm.at[pl.ds(base + 4, 4)], srcbb, sembs)
        pltpu.async_copy(dst_hbm.at[pl.ds(base + 4, 4)], dstbb, sembd)

        def chunk(g, state):
            rowa = base + g * 8
            pltpu.make_async_copy(src_hbm.at[pl.ds(rowa, 4)], srcba,
                                  semas).wait()
            pltpu.make_async_copy(dst_hbm.at[pl.ds(rowa, 4)], dstba,
                                  semad).wait()
            state = process4(srcba, dstba, rowa, state, lo)
            pltpu.async_copy(src_hbm.at[pl.ds(rowa + 8, 4)], srcba, semas)
            pltpu.async_copy(dst_hbm.at[pl.ds(rowa + 8, 4)], dstba, semad)
            rowb = rowa + 4
            pltpu.make_async_copy(src_hbm.at[pl.ds(rowb, 4)], srcbb,
                                  sembs).wait()
            pltpu.make_async_copy(dst_hbm.at[pl.ds(rowb, 4)], dstbb,
                                  sembd).wait()
            state = process4(srcbb, dstbb, rowb, state, lo)
            pltpu.async_copy(src_hbm.at[pl.ds(rowb + 8, 4)], srcbb, sembs)
            pltpu.async_copy(dst_hbm.at[pl.ds(rowb + 8, 4)], dstbb, sembd)
            return state

        state = lax.fori_loop(
            0, 49, chunk,
            (jnp.int32(0), jnp.int32(0), jnp.int32(0), jnp.int32(0)))
        pltpu.make_async_copy(src_hbm.at[pl.ds(base, 4)], srcba, semas).wait()
        pltpu.make_async_copy(dst_hbm.at[pl.ds(base, 4)], dstba, semad).wait()
        pltpu.make_async_copy(src_hbm.at[pl.ds(base, 4)], srcbb, sembs).wait()
        pltpu.make_async_copy(dst_hbm.at[pl.ds(base, 4)], dstbb, sembd).wait()
        q, pend, ip, sp = state
        f1 = pend == 1

        @pl.when(f1)
        def _():
            complete(ip, sp)

        sp1 = jnp.where(f1, 1, sp)
        np2 = jnp.where(f1, 1 - ip, 0)
        f2 = q >= 1

        @pl.when(f2)
        def _():
            issue(np2, lo)
            complete(np2, sp1)

        spf = jnp.where(f1 | f2, 1, sp)
        half = jnp.where(f2, np2, jnp.where(f1, ip, 1 - ip))

        @pl.when(spf == 1)
        def _():
            pltpu.make_async_copy(fbuf.at[pl.ds(half * FB, FB)],
                                  rst_sh.at[idxd2.at[half]], sem3).wait()

        plsc.subcore_barrier()

        r0p = s * 288
        for t in range(9):
            roff = r0p + t * 32
            pltpu.sync_copy(rst_sh.at[pl.ds(roff, 32)], rbuf)
            pltpu.sync_copy(fn_hbm.at[pl.ds(lo + roff, 32)], fidx32)

            def prow(i, carry2):
                for k in range(16):
                    v = rbuf[i, pl.ds(k * 16, 16)]
                    ev = jnp.exp(jnp.minimum(v, 0.0)) - 1.0
                    rbuf[i, pl.ds(k * 16, 16)] = jnp.where(v > 0.0, v, ev)
                return carry2

            lax.fori_loop(0, 32, prow, 0)
            pltpu.sync_copy(rbuf, pooled_sh.at[fidx32], add=True)
        plsc.subcore_barrier()
        return carry

    lax.fori_loop(0, NPASS, one_pass, 0)

    pltpu.sync_copy(pooled_sh.at[pl.ds(s * 4, 4)], pbuf)
    pltpu.sync_copy(pbuf, pp_hbm.at[c, pl.ds(s * 4, 4)])


def _b2(src2, dst2, feat, ee, rd, fn_pad):
    mesh = plsc.VectorSubcoreMesh(core_axis_name="c", subcore_axis_name="s",
                                  num_cores=NC, num_subcores=NS)
    f = pl.kernel(
        _b2_body,
        compiler_params=pltpu.CompilerParams(use_tc_tiling_on_sc=False, needs_layout_passes=False),
        out_type=jax.ShapeDtypeStruct((NC, G, HD), _f32),
        mesh=mesh,
        scratch_types=[
            pltpu.VMEM((4, 128), _i32),
            pltpu.VMEM((4, 128), _i32),
            pltpu.VMEM((4, 128), _i32),
            pltpu.VMEM((4, 128), _i32),
            pltpu.VMEM((QN,), _i32),
            pltpu.VMEM((QN,), _i32),
            pltpu.VMEM((QN,), _i32),
            pltpu.VMEM((2 * FB,), _i32),
            pltpu.VMEM((2 * FB,), _i32),
            pltpu.VMEM((2, FB), _i32),
            pltpu.VMEM((2 * FB,), _i32),
            pltpu.VMEM((2 * FB, HD), _f32),
            pltpu.VMEM((2 * FB, 16), _f32),
            pltpu.VMEM((2 * FB, 16), _f32),
            pltpu.VMEM((32, HD), _f32),
            pltpu.VMEM((32,), _i32),
            pltpu.VMEM((4, HD), _f32),
            pltpu.SemaphoreType.DMA,
            pltpu.SemaphoreType.DMA,
            pltpu.SemaphoreType.DMA,
            pltpu.SemaphoreType.DMA,
            pltpu.SemaphoreType.DMA,
            pltpu.SemaphoreType.DMA,
            pltpu.SemaphoreType.DMA,
            pltpu.SemaphoreType.DMA,
            pltpu.VMEM_SHARED((RNG + 16, HD), _f32),
            pltpu.VMEM_SHARED((G, HD), _f32),
        ],
    )
    return f(src2, dst2, feat, ee, rd, fn_pad)


def _tcc_body(fn_ref, pp_ref, bg_ref, wc_ref, bc_ref, out_ref, bge_ref,
              cnt_ref):
    i = pl.program_id(0)

    @pl.when(i == 0)
    def _():
        cnt_ref[...] = jnp.zeros((G, 128), _f32)

    ids = fn_ref[0]
    io = lax.broadcasted_iota(_i32, (2000, G), 1)
    oh = (ids == io).astype(_f32)
    ones = jnp.ones((2000, 1), _f32)
    cnt = lax.dot_general(oh, ones, (((0,), (0,)), ((), ())),
                          preferred_element_type=_f32)
    cnt_ref[:, 0:1] += cnt

    @pl.when(i == 24)
    def _():
        rc = 1.0 / jnp.maximum(cnt_ref[:, 0:1], 1.0)
        pooled = (pp_ref[0] + pp_ref[1]) * rc
        bio = lax.broadcasted_iota(_i32, (G, G), 1)
        ohg = (bg_ref[...] == bio).astype(_f32)
        bge = jnp.dot(ohg, pooled, preferred_element_type=_f32)
        bge_ref[...] = bge
        out_ref[...] = jnp.dot(bge, wc_ref[...],
                               preferred_element_type=_f32) + bc_ref[...]


def _tcc(fn_cols, pp, bg_col, wc, bc2):
    return pl.pallas_call(
        _tcc_body,
        grid=(25,),
        in_specs=[
            pl.BlockSpec((1, 2000, 1), lambda i: (i, 0, 0)),
            pl.BlockSpec((2, G, HD), lambda i: (0, 0, 0)),
            pl.BlockSpec((G, 1), lambda i: (0, 0)),
            pl.BlockSpec((HD, 2), lambda i: (0, 0)),
            pl.BlockSpec((1, 2), lambda i: (0, 0)),
        ],
        out_specs=[
            pl.BlockSpec((G, 2), lambda i: (0, 0)),
            pl.BlockSpec((G, HD), lambda i: (0, 0)),
        ],
        out_shape=[
            jax.ShapeDtypeStruct((G, 2), _f32),
            jax.ShapeDtypeStruct((G, HD), _f32),
        ],
        scratch_shapes=[pltpu.VMEM((G, 128), _f32)],
    )(fn_cols, pp, bg_col, wc, bc2)


@jax.jit
def kernel(x, edge_index, filename_ids, batched_g_ids, W, attn_l, attn_r,
           sa_W1, sa_b1, sa_W2, Wc, bc):
    x_pad = jnp.zeros((NP, 8), _f32).at[:N].set(x.astype(_f32))
    hrow = jnp.arange(HD, dtype=_i32) // D
    hcol = jnp.arange(H, dtype=_i32)
    sel = (hrow[:, None] == hcol[None, :]).astype(_f32)
    al = sel * attn_l.reshape(HD)[:, None]
    ar = sel * attn_r.reshape(HD)[:, None]

    src = edge_index[0].astype(_i32)
    dst = edge_index[1].astype(_i32)
    src2 = jnp.zeros((EP + 1024,), _i32).at[:E].set(src).reshape(ER + 8, 128)
    dst2 = jnp.full((EP + 1024,), PADDST, _i32).at[:E].set(dst).reshape(
        ER + 8, 128)

    fn_pad = jnp.zeros((55296,), _i32).at[:N].set(filename_ids.astype(_i32))
    fn_cols = filename_ids.astype(_i32).reshape(25, 2000, 1)
    bg_col = batched_g_ids.astype(_i32).reshape(G, 1)
    bc2 = bc.reshape(1, 2).astype(_f32)

    feat, el_t, er_t, cap = _tca(x_pad, W.astype(_f32), al, ar)
    ee, dp = _b1(src2, dst2, el_t, er_t, cap)
    rd = _a2(dp)
    pp = _b2(src2, dst2, feat, ee, rd, fn_pad)
    out, bge = _tcc(fn_cols, pp, bg_col, Wc.astype(_f32), bc2)
    return (out, bge)


# speedup vs baseline: 1.0550x; 1.0550x over previous
import functools

import jax
import jax.numpy as jnp
from jax import lax
from jax.experimental import pallas as pl
from jax.experimental.pallas import tpu as pltpu
from jax.experimental.pallas import tpu_sc as plsc

N = 50000
E = 800000
H = 8
D = 32
HD = H * D
G = 64

NC, NS, LN = 2, 16, 16

BN = 2048
NB = 25
NP = BN * NB
EP = 802816
ER = EP // 128
RNG = 4608
NPASS = 6
DUMP = RNG
NT = 50240
NCOV = 50176
PADDST = 50176
QN = 256
FB = 64

_f32 = jnp.float32
_i32 = jnp.int32


def _tca_body(x_ref, w_ref, al_ref, ar_ref, feat_ref, el_ref, er_ref, cap_ref,
              acc_ref):
    i = pl.program_id(0)
    x = x_ref[...]
    feat = jnp.dot(x, w_ref[...], preferred_element_type=_f32)
    feat_ref[...] = feat
    el = jnp.dot(feat, al_ref[...], preferred_element_type=_f32)
    er = jnp.dot(feat, ar_ref[...], preferred_element_type=_f32)
    z = jnp.zeros_like(el)
    el_ref[...] = jnp.concatenate([el, z], axis=1)
    er_ref[...] = jnp.concatenate([er, z], axis=1)
    bm = jnp.concatenate([jnp.max(el, axis=0, keepdims=True),
                          jnp.max(er, axis=0, keepdims=True)], axis=1)

    @pl.when(i == 0)
    def _():
        acc_ref[...] = jnp.full((1, 16), -jnp.inf, _f32)
        cap_ref[...] = jnp.zeros((8, 128), _f32)

    acc = jnp.maximum(acc_ref[...], bm)
    acc_ref[...] = acc

    @pl.when(i == NB - 1)
    def _():
        csum = acc[:, 0:8] + acc[:, 8:16]
        cap = jnp.maximum(csum, 0.2 * csum)
        big = jnp.full((1, 8), 1e30, _f32)
        cap_ref[0:1, 0:16] = jnp.concatenate([cap, big], axis=1)


def _tca(x_pad, w, al, ar):
    return pl.pallas_call(
        _tca_body,
        grid=(NB,),
        in_specs=[
            pl.BlockSpec((BN, 8), lambda i: (i, 0)),
            pl.BlockSpec((8, HD), lambda i: (0, 0)),
            pl.BlockSpec((HD, 8), lambda i: (0, 0)),
            pl.BlockSpec((HD, 8), lambda i: (0, 0)),
        ],
        out_specs=[
            pl.BlockSpec((BN, HD), lambda i: (i, 0)),
            pl.BlockSpec((BN, 16), lambda i: (i, 0)),
            pl.BlockSpec((BN, 16), lambda i: (i, 0)),
            pl.BlockSpec((8, 128), lambda i: (0, 0)),
        ],
        out_shape=[
            jax.ShapeDtypeStruct((NP, HD), _f32),
            jax.ShapeDtypeStruct((NP, 16), _f32),
            jax.ShapeDtypeStruct((NP, 16), _f32),
            jax.ShapeDtypeStruct((8, 128), _f32),
        ],
        scratch_shapes=[pltpu.VMEM((1, 16), _f32)],
    )(x_pad, w, al, ar)


def _b1_body(src_hbm, dst_hbm, el_hbm, er_hbm, cap_hbm, ee_hbm, dp_hbm,
             srcb, dstb, elrows, errows, eerows, cvec, way,
             sem0, sem1, semw, sems, denom_sh):
    c = lax.axis_index("c")
    s = lax.axis_index("s")

    def zrow(i, carry):
        way[i, :] = jnp.zeros((16,), _f32)
        return carry

    lax.fori_loop(0, 785, zrow, 0)
    for t in range(4):
        pltpu.sync_copy(way, denom_sh.at[pl.ds(s * 3140 + t * 785, 785)])
    pltpu.sync_copy(cap_hbm.at[0, pl.ds(0, 16)], cvec)
    plsc.subcore_barrier()

    base_row = (c * NS + s) * 196

    def chunk(g, carry):
        ph = lax.rem(g, 2)
        row = base_row + g * 4

        @pl.when(g >= 2)
        def _():
            pltpu.make_async_copy(
                eerows.at[pl.ds(ph * 512, 512)],
                ee_hbm.at[pl.ds(row * 128, 512)], semw).wait()
            for j in range(4):
                pltpu.make_async_copy(
                    eerows.at[pl.ds(ph * 512 + j * 128, 128)],
                    denom_sh.at[dstb.at[ph * 4 + j]], sems).wait()

        pltpu.sync_copy(src_hbm.at[pl.ds(row, 4)],
                        srcb.at[pl.ds(ph * 4, 4)])
        pltpu.sync_copy(dst_hbm.at[pl.ds(row, 4)],
                        dstb.at[pl.ds(ph * 4, 4)])
        cps = []
        for j in range(4):
            cps.append(pltpu.async_copy(
                el_hbm.at[srcb.at[ph * 4 + j]],
                elrows.at[pl.ds(j * 128, 128)], sem0))
            cps.append(pltpu.async_copy(
                er_hbm.at[dstb.at[ph * 4 + j]],
                errows.at[pl.ds(j * 128, 128)], sem1))
        for cp in cps:
            cp.wait()
        cv = cvec[...]

        def edge(i, carry2):
            v = elrows[i, :] + errows[i, :]
            v = jnp.maximum(v, 0.2 * v)
            eerows[ph * 512 + i, :] = jnp.exp(v - cv)
            return carry2

        lax.fori_loop(0, 512, edge, 0, unroll=4)
        pltpu.async_copy(eerows.at[pl.ds(ph * 512, 512)],
                         ee_hbm.at[pl.ds(row * 128, 512)], semw)
        for j in range(4):
            pltpu.async_copy(eerows.at[pl.ds(ph * 512 + j * 128, 128)],
                             denom_sh.at[dstb.at[ph * 4 + j]], sems,
                             add=True)
        return carry

    lax.fori_loop(0, 49, chunk, 0)
    for _ in range(2):
        pltpu.make_async_copy(eerows.at[pl.ds(0, 512)],
                              ee_hbm.at[pl.ds(0, 512)], semw).wait()
        for j in range(4):
            pltpu.make_async_copy(eerows.at[pl.ds(j * 128, 128)],
                                  denom_sh.at[dstb.at[j]], sems).wait()
    plsc.subcore_barrier()
    for t in range(4):
        pltpu.sync_copy(denom_sh.at[pl.ds(s * 3140 + t * 785, 785)], way)
        pltpu.sync_copy(way, dp_hbm.at[c, pl.ds(s * 3140 + t * 785, 785)])


def _b1(src2, dst2, el_t, er_t, cap):
    mesh = plsc.VectorSubcoreMesh(core_axis_name="c", subcore_axis_name="s",
                                  num_cores=NC, num_subcores=NS)
    f = pl.kernel(
        _b1_body,
        compiler_params=pltpu.CompilerParams(use_tc_tiling_on_sc=False, needs_layout_passes=False),
        out_type=(jax.ShapeDtypeStruct((EP, 16), _f32),
                  jax.ShapeDtypeStruct((NC, NT, 16), _f32)),
        mesh=mesh,
        scratch_types=[
            pltpu.VMEM((8, 128), _i32),
            pltpu.VMEM((8, 128), _i32),
            pltpu.VMEM((512, 16), _f32),
            pltpu.VMEM((512, 16), _f32),
            pltpu.VMEM((1024, 16), _f32),
            pltpu.VMEM((16,), _f32),
            pltpu.VMEM((785, 16), _f32),
            pltpu.SemaphoreType.DMA,
            pltpu.SemaphoreType.DMA,
            pltpu.SemaphoreType.DMA,
            pltpu.SemaphoreType.DMA,
            pltpu.VMEM_SHARED((NT, 16), _f32),
        ],
    )
    return f(src2, dst2, el_t, er_t, cap)


def _a2_body(dp_ref, rd_ref):
    d = dp_ref[0] + dp_ref[1]
    rd_ref[...] = 1.0 / jnp.maximum(d, 1e-30)


def _a2(dp):
    return pl.pallas_call(
        _a2_body,
        grid=(8,),
        in_specs=[pl.BlockSpec((2, 6280, 16), lambda i: (0, i, 0))],
        out_specs=pl.BlockSpec((6280, 16), lambda i: (i, 0)),
        out_shape=jax.ShapeDtypeStruct((NT, 16), _f32),
    )(dp)


def _b2_body(src_hbm, dst_hbm, feat_hbm, ee_hbm, rd_hbm, fn_hbm, pp_hbm,
             srcba, dstba, srcbb, dstbb, qsrc, qdl, qpos, idxs, idxp, idxd2,
             idxr, fbuf, eebuf, rdbuf, rbuf, fidx32, pbuf,
             sem0, sem1, sem2, sem3, semas, semad, sembs, sembd, rst_sh,
             pooled_sh):
    c = lax.axis_index("c")
    s = lax.axis_index("s")
    iota16 = lax.broadcasted_iota(_i32, (16,), 0)
    zero16 = jnp.zeros((16,), _f32)

    def zrow(i, carry):
        for k in range(16):
            rbuf[i, pl.ds(k * 16, 16)] = zero16
        return carry

    lax.fori_loop(0, 32, zrow, 0)
    pltpu.sync_copy(rbuf.at[pl.ds(0, 4)], pooled_sh.at[pl.ds(s * 4, 4)])

    dump16 = jnp.full((16,), DUMP, _i32)
    z16i = jnp.zeros((16,), _i32)

    def queue_reset():
        for k in range(QN // 16):
            qsrc[pl.ds(k * 16, 16)] = z16i
            qdl[pl.ds(k * 16, 16)] = dump16
            qpos[pl.ds(k * 16, 16)] = z16i

    def issue(np, lo):
        for k in range(FB // 16):
            sv = qsrc[pl.ds(k * 16, 16)]
            pv = qpos[pl.ds(k * 16, 16)]
            dv = qdl[pl.ds(k * 16, 16)]
            idxs[pl.ds(np * FB + k * 16, 16)] = sv
            idxp[pl.ds(np * FB + k * 16, 16)] = pv
            idxd2[np, pl.ds(k * 16, 16)] = dv
            idxr[pl.ds(np * FB + k * 16, 16)] = jnp.minimum(dv + lo, NT - 1)
        pltpu.async_copy(feat_hbm.at[idxs.at[pl.ds(np * FB, FB)]],
                         fbuf.at[pl.ds(np * FB, FB)], sem0)
        pltpu.async_copy(ee_hbm.at[idxp.at[pl.ds(np * FB, FB)]],
                         eebuf.at[pl.ds(np * FB, FB)], sem1)
        pltpu.async_copy(rd_hbm.at[idxr.at[pl.ds(np * FB, FB)]],
                         rdbuf.at[pl.ds(np * FB, FB)], sem2)
        for k in range((QN - FB) // 16):
            qsrc[pl.ds(k * 16, 16)] = qsrc[pl.ds(FB + k * 16, 16)]
            qdl[pl.ds(k * 16, 16)] = qdl[pl.ds(FB + k * 16, 16)]
            qpos[pl.ds(k * 16, 16)] = qpos[pl.ds(FB + k * 16, 16)]
        for k in range(FB // 16):
            qsrc[pl.ds(QN - FB + k * 16, 16)] = z16i
            qdl[pl.ds(QN - FB + k * 16, 16)] = dump16
            qpos[pl.ds(QN - FB + k * 16, 16)] = z16i

    def complete(ip, spin):
        @pl.when(spin == 1)
        def _():
            pltpu.make_async_copy(fbuf.at[pl.ds((1 - ip) * FB, FB)],
                                  rst_sh.at[idxd2.at[1 - ip]], sem3).wait()

        pltpu.make_async_copy(feat_hbm.at[idxs.at[pl.ds(ip * FB, FB)]],
                              fbuf.at[pl.ds(ip * FB, FB)], sem0).wait()
        pltpu.make_async_copy(ee_hbm.at[idxp.at[pl.ds(ip * FB, FB)]],
                              eebuf.at[pl.ds(ip * FB, FB)], sem1).wait()
        pltpu.make_async_copy(rd_hbm.at[idxr.at[pl.ds(ip * FB, FB)]],
                              rdbuf.at[pl.ds(ip * FB, FB)], sem2).wait()

        def mrow(i, carry):
            r = ip * FB + i
            iv = jnp.zeros((16,), _i32) + r
            for h in range(8):
                hv = jnp.full((16,), h, _i32)
                asp = (plsc.load_gather(eebuf, [iv, hv])
                       * plsc.load_gather(rdbuf, [iv, hv]))
                for cc in (2 * h, 2 * h + 1):
                    fbuf[r, pl.ds(cc * 16, 16)] = (
                        fbuf[r, pl.ds(cc * 16, 16)] * asp)
            return carry

        lax.fori_loop(0, FB, mrow, 0)
        pltpu.async_copy(fbuf.at[pl.ds(ip * FB, FB)],
                         rst_sh.at[idxd2.at[ip]], sem3, add=True)

    def maybe_flush(state, lo, thresh):
        q, pend, ip, sp = state
        hit = q >= thresh

        @pl.when(hit & (pend == 1))
        def _():
            complete(ip, sp)

        np = jnp.where(pend == 1, 1 - ip, 0)

        @pl.when(hit)
        def _():
            issue(np, lo)

        q = jnp.where(hit, q - FB, q)
        sp = jnp.where(hit & (pend == 1), 1, sp)
        pend = jnp.where(hit, 1, pend)
        ip = jnp.where(hit, np, ip)
        return (q, pend, ip, sp)

    def one_pass(p, carry):
        lo = (c * NPASS + p) * RNG
        lax.fori_loop(0, 32, zrow, 0)
        r0 = s * 289
        for t in range(9):
            pltpu.sync_copy(rbuf, rst_sh.at[pl.ds(r0 + t * 32, 32)])
        pltpu.sync_copy(rbuf.at[pl.ds(0, 1)], rst_sh.at[pl.ds(r0 + 288, 1)])
        queue_reset()
        plsc.subcore_barrier()

        hi = jnp.minimum(lo + RNG, NCOV)

        def process4(sb, db, row, state, lo):
            for j in range(4):
                q = state[0]
                for gg in range(8):
                    dv = db[j, pl.ds(gg * 16, 16)]
                    sv = sb[j, pl.ds(gg * 16, 16)]
                    mask = (dv >= lo) & (dv < hi)
                    m01 = jnp.where(mask, 1, 0).astype(_i32)
                    csum = plsc.cumsum(m01)
                    tgt = q + csum - 1
                    plsc.store_scatter(qsrc, [tgt], sv, mask=mask)
                    plsc.store_scatter(qdl, [tgt], dv - lo, mask=mask)
                    pos = (row + j) * 128 + gg * 16 + iota16
                    plsc.store_scatter(qpos, [tgt], pos, mask=mask)
                    q = q + jnp.sum(m01)
                state = (q,) + state[1:]
                state = maybe_flush(state, lo, FB)
            return state

        base = s * 392
        pltpu.async_copy(src_hbm.at[pl.ds(base, 4)], srcba, semas)
        pltpu.async_copy(dst_hbm.at[pl.ds(base, 4)], dstba, semad)
        pltpu.async_copy(src_hbm.at[pl.ds(base + 4, 4)], srcbb, sembs)
        pltpu.async_copy(dst_hbm.at[pl.ds(base + 4, 4)], dstbb, sembd)

        def chunk(g, state):
            rowa = base + g * 8
            pltpu.make_async_copy(src_hbm.at[pl.ds(rowa, 4)], srcba,
                                  semas).wait()
            pltpu.make_async_copy(dst_hbm.at[pl.ds(rowa, 4)], dstba,
                                  semad).wait()
            state = process4(srcba, dstba, rowa, state, lo)
            pltpu.async_copy(src_hbm.at[pl.ds(rowa + 8, 4)], srcba, semas)
            pltpu.async_copy(dst_hbm.at[pl.ds(rowa + 8, 4)], dstba, semad)
            rowb = rowa + 4
            pltpu.make_async_copy(src_hbm.at[pl.ds(rowb, 4)], srcbb,
                                  sembs).wait()
            pltpu.make_async_copy(dst_hbm.at[pl.ds(rowb, 4)], dstbb,
                                  sembd).wait()
            state = process4(srcbb, dstbb, rowb, state, lo)
            pltpu.async_copy(src_hbm.at[pl.ds(rowb + 8, 4)], srcbb, sembs)
            pltpu.async_copy(dst_hbm.at[pl.ds(rowb + 8, 4)], dstbb, sembd)
            return state

        state = lax.fori_loop(
            0, 49, chunk,
            (jnp.int32(0), jnp.int32(0), jnp.int32(0), jnp.int32(0)))
        pltpu.make_async_copy(src_hbm.at[pl.ds(base, 4)], srcba, semas).wait()
        pltpu.make_async_copy(dst_hbm.at[pl.ds(base, 4)], dstba, semad).wait()
        pltpu.make_async_copy(src_hbm.at[pl.ds(base, 4)], srcbb, sembs).wait()
        pltpu.make_async_copy(dst_hbm.at[pl.ds(base, 4)], dstbb, sembd).wait()
        q, pend, ip, sp = state
        f1 = pend == 1

        @pl.when(f1)
        def _():
            complete(ip, sp)

        sp1 = jnp.where(f1, 1, sp)
        np2 = jnp.where(f1, 1 - ip, 0)
        f2 = q >= 1

        @pl.when(f2)
        def _():
            issue(np2, lo)
            complete(np2, sp1)

        spf = jnp.where(f1 | f2, 1, sp)
        half = jnp.where(f2, np2, jnp.where(f1, ip, 1 - ip))

        @pl.when(spf == 1)
        def _():
            pltpu.make_async_copy(fbuf.at[pl.ds(half * FB, FB)],
                                  rst_sh.at[idxd2.at[half]], sem3).wait()

        plsc.subcore_barrier()

        r0p = s * 288
        for t in range(9):
            roff = r0p + t * 32
            pltpu.sync_copy(rst_sh.at[pl.ds(roff, 32)], rbuf)
            pltpu.sync_copy(fn_hbm.at[pl.ds(lo + roff, 32)], fidx32)

            def prow(i, carry2):
                for k in range(16):
                    v = rbuf[i, pl.ds(k * 16, 16)]
                    ev = jnp.exp(jnp.minimum(v, 0.0)) - 1.0
                    rbuf[i, pl.ds(k * 16, 16)] = jnp.where(v > 0.0, v, ev)
                return carry2

            lax.fori_loop(0, 32, prow, 0)
            pltpu.sync_copy(rbuf, pooled_sh.at[fidx32], add=True)
        plsc.subcore_barrier()
        return carry

    lax.fori_loop(0, NPASS, one_pass, 0)

    pltpu.sync_copy(pooled_sh.at[pl.ds(s * 4, 4)], pbuf)
    pltpu.sync_copy(pbuf, pp_hbm.at[c, pl.ds(s * 4, 4)])


def _b2(src2, dst2, feat, ee, rd, fn_pad):
    mesh = plsc.VectorSubcoreMesh(core_axis_name="c", subcore_axis_name="s",
                                  num_cores=NC, num_subcores=NS)
    f = pl.kernel(
        _b2_body,
        compiler_params=pltpu.CompilerParams(use_tc_tiling_on_sc=False, needs_layout_passes=False),
        out_type=jax.ShapeDtypeStruct((NC, G, HD), _f32),
        mesh=mesh,
        scratch_types=[
            pltpu.VMEM((4, 128), _i32),
            pltpu.VMEM((4, 128), _i32),
            pltpu.VMEM((4, 128), _i32),
            pltpu.VMEM((4, 128), _i32),
            pltpu.VMEM((QN,), _i32),
            pltpu.VMEM((QN,), _i32),
            pltpu.VMEM((QN,), _i32),
            pltpu.VMEM((2 * FB,), _i32),
            pltpu.VMEM((2 * FB,), _i32),
            pltpu.VMEM((2, FB), _i32),
            pltpu.VMEM((2 * FB,), _i32),
            pltpu.VMEM((2 * FB, HD), _f32),
            pltpu.VMEM((2 * FB, 16), _f32),
            pltpu.VMEM((2 * FB, 16), _f32),
            pltpu.VMEM((32, HD), _f32),
            pltpu.VMEM((32,), _i32),
            pltpu.VMEM((4, HD), _f32),
            pltpu.SemaphoreType.DMA,
            pltpu.SemaphoreType.DMA,
            pltpu.SemaphoreType.DMA,
            pltpu.SemaphoreType.DMA,
            pltpu.SemaphoreType.DMA,
            pltpu.SemaphoreType.DMA,
            pltpu.SemaphoreType.DMA,
            pltpu.SemaphoreType.DMA,
            pltpu.VMEM_SHARED((RNG + 16, HD), _f32),
            pltpu.VMEM_SHARED((G, HD), _f32),
        ],
    )
    return f(src2, dst2, feat, ee, rd, fn_pad)


def _tcc_body(fn_ref, pp_ref, bg_ref, wc_ref, bc_ref, out_ref, bge_ref,
              cnt_ref):
    i = pl.program_id(0)

    @pl.when(i == 0)
    def _():
        cnt_ref[...] = jnp.zeros((G, 128), _f32)

    ids = fn_ref[0]
    io = lax.broadcasted_iota(_i32, (2000, G), 1)
    oh = (ids == io).astype(_f32)
    ones = jnp.ones((2000, 1), _f32)
    cnt = lax.dot_general(oh, ones, (((0,), (0,)), ((), ())),
                          preferred_element_type=_f32)
    cnt_ref[:, 0:1] += cnt

    @pl.when(i == 24)
    def _():
        rc = 1.0 / jnp.maximum(cnt_ref[:, 0:1], 1.0)
        pooled = (pp_ref[0] + pp_ref[1]) * rc
        bio = lax.broadcasted_iota(_i32, (G, G), 1)
        ohg = (bg_ref[...] == bio).astype(_f32)
        bge = jnp.dot(ohg, pooled, preferred_element_type=_f32)
        bge_ref[...] = bge
        out_ref[...] = jnp.dot(bge, wc_ref[...],
                               preferred_element_type=_f32) + bc_ref[...]


def _tcc(fn_cols, pp, bg_col, wc, bc2):
    return pl.pallas_call(
        _tcc_body,
        grid=(25,),
        in_specs=[
            pl.BlockSpec((1, 2000, 1), lambda i: (i, 0, 0)),
            pl.BlockSpec((2, G, HD), lambda i: (0, 0, 0)),
            pl.BlockSpec((G, 1), lambda i: (0, 0)),
            pl.BlockSpec((HD, 2), lambda i: (0, 0)),
            pl.BlockSpec((1, 2), lambda i: (0, 0)),
        ],
        out_specs=[
            pl.BlockSpec((G, 2), lambda i: (0, 0)),
            pl.BlockSpec((G, HD), lambda i: (0, 0)),
        ],
        out_shape=[
            jax.ShapeDtypeStruct((G, 2), _f32),
            jax.ShapeDtypeStruct((G, HD), _f32),
        ],
        scratch_shapes=[pltpu.VMEM((G, 128), _f32)],
    )(fn_cols, pp, bg_col, wc, bc2)


@jax.jit
def kernel(x, edge_index, filename_ids, batched_g_ids, W, attn_l, attn_r,
           sa_W1, sa_b1, sa_W2, Wc, bc):
    x_pad = jnp.zeros((NP, 8), _f32).at[:N].set(x.astype(_f32))
    hrow = jnp.arange(HD, dtype=_i32) // D
    hcol = jnp.arange(H, dtype=_i32)
    sel = (hrow[:, None] == hcol[None, :]).astype(_f32)
    al = sel * attn_l.reshape(HD)[:, None]
    ar = sel * attn_r.reshape(HD)[:, None]

    src = edge_index[0].astype(_i32)
    dst = edge_index[1].astype(_i32)
    src2 = jnp.zeros((EP + 1024,), _i32).at[:E].set(src).reshape(ER + 8, 128)
    dst2 = jnp.full((EP + 1024,), PADDST, _i32).at[:E].set(dst).reshape(
        ER + 8, 128)

    fn_pad = jnp.zeros((55296,), _i32).at[:N].set(filename_ids.astype(_i32))
    fn_cols = filename_ids.astype(_i32).reshape(25, 2000, 1)
    bg_col = batched_g_ids.astype(_i32).reshape(G, 1)
    bc2 = bc.reshape(1, 2).astype(_f32)

    feat, el_t, er_t, cap = _tca(x_pad, W.astype(_f32), al, ar)
    ee, dp = _b1(src2, dst2, el_t, er_t, cap)
    rd = _a2(dp)
    pp = _b2(src2, dst2, feat, ee, rd, fn_pad)
    out, bge = _tcc(fn_cols, pp, bg_col, Wc.astype(_f32), bc2)
    return (out, bge)


# speedup vs baseline: 1.0757x; 1.0196x over previous
import functools

import jax
import jax.numpy as jnp
from jax import lax
from jax.experimental import pallas as pl
from jax.experimental.pallas import tpu as pltpu
from jax.experimental.pallas import tpu_sc as plsc

N = 50000
E = 800000
H = 8
D = 32
HD = H * D
G = 64

NC, NS, LN = 2, 16, 16

BN = 2048
NB = 25
NP = BN * NB
EP = 802816
ER = EP // 128
RNG = 4608
NPASS = 6
DUMP = RNG
NT = 50240
NCOV = 50176
PADDST = 50176
QN = 256
FB = 64

_f32 = jnp.float32
_i32 = jnp.int32


def _tca_body(x_ref, w_ref, al_ref, ar_ref, feat_ref, el_ref, er_ref, cap_ref,
              acc_ref):
    i = pl.program_id(0)
    x = x_ref[...]
    feat = jnp.dot(x, w_ref[...], preferred_element_type=_f32)
    feat_ref[...] = feat
    el = jnp.dot(feat, al_ref[...], preferred_element_type=_f32)
    er = jnp.dot(feat, ar_ref[...], preferred_element_type=_f32)
    z = jnp.zeros_like(el)
    el_ref[...] = jnp.concatenate([el, z], axis=1)
    er_ref[...] = jnp.concatenate([er, z], axis=1)
    bm = jnp.concatenate([jnp.max(el, axis=0, keepdims=True),
                          jnp.max(er, axis=0, keepdims=True)], axis=1)

    @pl.when(i == 0)
    def _():
        acc_ref[...] = jnp.full((1, 16), -jnp.inf, _f32)
        cap_ref[...] = jnp.zeros((8, 128), _f32)

    acc = jnp.maximum(acc_ref[...], bm)
    acc_ref[...] = acc

    @pl.when(i == NB - 1)
    def _():
        csum = acc[:, 0:8] + acc[:, 8:16]
        cap = jnp.maximum(csum, 0.2 * csum)
        big = jnp.full((1, 8), 1e30, _f32)
        cap_ref[0:1, 0:16] = jnp.concatenate([cap, big], axis=1)


def _tca(x_pad, w, al, ar):
    return pl.pallas_call(
        _tca_body,
        grid=(NB,),
        in_specs=[
            pl.BlockSpec((BN, 8), lambda i: (i, 0)),
            pl.BlockSpec((8, HD), lambda i: (0, 0)),
            pl.BlockSpec((HD, 8), lambda i: (0, 0)),
            pl.BlockSpec((HD, 8), lambda i: (0, 0)),
        ],
        out_specs=[
            pl.BlockSpec((BN, HD), lambda i: (i, 0)),
            pl.BlockSpec((BN, 16), lambda i: (i, 0)),
            pl.BlockSpec((BN, 16), lambda i: (i, 0)),
            pl.BlockSpec((8, 128), lambda i: (0, 0)),
        ],
        out_shape=[
            jax.ShapeDtypeStruct((NP, HD), _f32),
            jax.ShapeDtypeStruct((NP, 16), _f32),
            jax.ShapeDtypeStruct((NP, 16), _f32),
            jax.ShapeDtypeStruct((8, 128), _f32),
        ],
        scratch_shapes=[pltpu.VMEM((1, 16), _f32)],
    )(x_pad, w, al, ar)


def _b1_body(src_hbm, dst_hbm, el_hbm, er_hbm, cap_hbm, ee_hbm, dp_hbm,
             srcb, dstb, elrows, errows, eerows, cvec, way,
             semga, semgb, semha, semhb, semw, sems, denom_sh):
    c = lax.axis_index("c")
    s = lax.axis_index("s")

    def zrow(i, carry):
        way[i, :] = jnp.zeros((16,), _f32)
        return carry

    lax.fori_loop(0, 785, zrow, 0)
    for t in range(4):
        pltpu.sync_copy(way, denom_sh.at[pl.ds(s * 3140 + t * 785, 785)])
    pltpu.sync_copy(cap_hbm.at[0, pl.ds(0, 16)], cvec)
    plsc.subcore_barrier()

    base_row = (c * NS + s) * 196

    def fetch(g, ph):
        row = base_row + g * 4
        pltpu.sync_copy(src_hbm.at[pl.ds(row, 4)],
                        srcb.at[pl.ds(ph * 4, 4)])
        pltpu.sync_copy(dst_hbm.at[pl.ds(row, 4)],
                        dstb.at[pl.ds(ph * 4, 4)])
        for j in range(4):
            pltpu.async_copy(el_hbm.at[srcb.at[ph * 4 + j]],
                             elrows.at[pl.ds(ph * 512 + j * 128, 128)],
                             semga if ph == 0 else semgb)
            pltpu.async_copy(er_hbm.at[dstb.at[ph * 4 + j]],
                             errows.at[pl.ds(ph * 512 + j * 128, 128)],
                             semha if ph == 0 else semhb)

    fetch(0, 0)

    def chunk2(g2, carry):
        for ph in range(2):
            g = g2 * 2 + ph
            row = base_row + g * 4

            @pl.when(g >= 2)
            def _():
                pltpu.make_async_copy(
                    eerows.at[pl.ds(ph * 512, 512)],
                    ee_hbm.at[pl.ds(row * 128, 512)], semw).wait()
                for j in range(4):
                    pltpu.make_async_copy(
                        eerows.at[pl.ds(ph * 512 + j * 128, 128)],
                        denom_sh.at[dstb.at[ph * 4 + j]], sems).wait()

            @pl.when(g < 48)
            def _():
                fetch(g + 1, 1 - ph)

            for j in range(4):
                pltpu.make_async_copy(
                    el_hbm.at[srcb.at[ph * 4 + j]],
                    elrows.at[pl.ds(ph * 512 + j * 128, 128)],
                    semga if ph == 0 else semgb).wait()
                pltpu.make_async_copy(
                    er_hbm.at[dstb.at[ph * 4 + j]],
                    errows.at[pl.ds(ph * 512 + j * 128, 128)],
                    semha if ph == 0 else semhb).wait()
            cv = cvec[...]

            def edge(i, carry2):
                v = (elrows[ph * 512 + i, :] + errows[ph * 512 + i, :])
                v = jnp.maximum(v, 0.2 * v)
                eerows[ph * 512 + i, :] = jnp.exp(v - cv)
                return carry2

            lax.fori_loop(0, 512, edge, 0, unroll=4)
            pltpu.async_copy(eerows.at[pl.ds(ph * 512, 512)],
                             ee_hbm.at[pl.ds(row * 128, 512)], semw)
            for j in range(4):
                pltpu.async_copy(
                    eerows.at[pl.ds(ph * 512 + j * 128, 128)],
                    denom_sh.at[dstb.at[ph * 4 + j]], sems, add=True)
        return carry

    lax.fori_loop(0, 24, chunk2, 0)
    def chunk48():
        g = 48
        ph = 0
        row = base_row + g * 4
        pltpu.make_async_copy(
            eerows.at[pl.ds(ph * 512, 512)],
            ee_hbm.at[pl.ds(row * 128, 512)], semw).wait()
        for j in range(4):
            pltpu.make_async_copy(
                eerows.at[pl.ds(ph * 512 + j * 128, 128)],
                denom_sh.at[dstb.at[ph * 4 + j]], sems).wait()
        for j in range(4):
            pltpu.make_async_copy(
                el_hbm.at[srcb.at[ph * 4 + j]],
                elrows.at[pl.ds(ph * 512 + j * 128, 128)], semga).wait()
            pltpu.make_async_copy(
                er_hbm.at[dstb.at[ph * 4 + j]],
                errows.at[pl.ds(ph * 512 + j * 128, 128)], semha).wait()
        cv = cvec[...]

        def edge(i, carry2):
            v = elrows[ph * 512 + i, :] + errows[ph * 512 + i, :]
            v = jnp.maximum(v, 0.2 * v)
            eerows[ph * 512 + i, :] = jnp.exp(v - cv)
            return carry2

        lax.fori_loop(0, 512, edge, 0, unroll=4)
        pltpu.async_copy(eerows.at[pl.ds(ph * 512, 512)],
                         ee_hbm.at[pl.ds(row * 128, 512)], semw)
        for j in range(4):
            pltpu.async_copy(eerows.at[pl.ds(ph * 512 + j * 128, 128)],
                             denom_sh.at[dstb.at[ph * 4 + j]], sems,
                             add=True)

    chunk48()
    for _ in range(2):
        pltpu.make_async_copy(eerows.at[pl.ds(0, 512)],
                              ee_hbm.at[pl.ds(0, 512)], semw).wait()
        for j in range(4):
            pltpu.make_async_copy(eerows.at[pl.ds(j * 128, 128)],
                                  denom_sh.at[dstb.at[j]], sems).wait()
    plsc.subcore_barrier()
    for t in range(4):
        pltpu.sync_copy(denom_sh.at[pl.ds(s * 3140 + t * 785, 785)], way)
        pltpu.sync_copy(way, dp_hbm.at[c, pl.ds(s * 3140 + t * 785, 785)])


def _b1(src2, dst2, el_t, er_t, cap):
    mesh = plsc.VectorSubcoreMesh(core_axis_name="c", subcore_axis_name="s",
                                  num_cores=NC, num_subcores=NS)
    f = pl.kernel(
        _b1_body,
        compiler_params=pltpu.CompilerParams(use_tc_tiling_on_sc=False, needs_layout_passes=False),
        out_type=(jax.ShapeDtypeStruct((EP, 16), _f32),
                  jax.ShapeDtypeStruct((NC, NT, 16), _f32)),
        mesh=mesh,
        scratch_types=[
            pltpu.VMEM((8, 128), _i32),
            pltpu.VMEM((8, 128), _i32),
            pltpu.VMEM((1024, 16), _f32),
            pltpu.VMEM((1024, 16), _f32),
            pltpu.VMEM((1024, 16), _f32),
            pltpu.VMEM((16,), _f32),
            pltpu.VMEM((785, 16), _f32),
            pltpu.SemaphoreType.DMA,
            pltpu.SemaphoreType.DMA,
            pltpu.SemaphoreType.DMA,
            pltpu.SemaphoreType.DMA,
            pltpu.SemaphoreType.DMA,
            pltpu.SemaphoreType.DMA,
            pltpu.VMEM_SHARED((NT, 16), _f32),
        ],
    )
    return f(src2, dst2, el_t, er_t, cap)


def _a2_body(dp_ref, rd_ref):
    d = dp_ref[0] + dp_ref[1]
    rd_ref[...] = 1.0 / jnp.maximum(d, 1e-30)


def _a2(dp):
    return pl.pallas_call(
        _a2_body,
        grid=(8,),
        in_specs=[pl.BlockSpec((2, 6280, 16), lambda i: (0, i, 0))],
        out_specs=pl.BlockSpec((6280, 16), lambda i: (i, 0)),
        out_shape=jax.ShapeDtypeStruct((NT, 16), _f32),
    )(dp)


def _b2_body(src_hbm, dst_hbm, feat_hbm, ee_hbm, rd_hbm, fn_hbm, pp_hbm,
             srcba, dstba, srcbb, dstbb, qsrc, qdl, qpos, idxs, idxp, idxd2,
             idxr, fbuf, eebuf, rdbuf, rbuf, fidx32, pbuf,
             sem0, sem1, sem2, sem3, semas, semad, sembs, sembd, rst_sh,
             pooled_sh):
    c = lax.axis_index("c")
    s = lax.axis_index("s")
    iota16 = lax.broadcasted_iota(_i32, (16,), 0)
    zero16 = jnp.zeros((16,), _f32)

    def zrow(i, carry):
        for k in range(16):
            rbuf[i, pl.ds(k * 16, 16)] = zero16
        return carry

    lax.fori_loop(0, 32, zrow, 0)
    pltpu.sync_copy(rbuf.at[pl.ds(0, 4)], pooled_sh.at[pl.ds(s * 4, 4)])

    dump16 = jnp.full((16,), DUMP, _i32)
    z16i = jnp.zeros((16,), _i32)

    def queue_reset():
        for k in range(QN // 16):
            qsrc[pl.ds(k * 16, 16)] = z16i
            qdl[pl.ds(k * 16, 16)] = dump16
            qpos[pl.ds(k * 16, 16)] = z16i

    def issue(np, lo):
        for k in range(FB // 16):
            sv = qsrc[pl.ds(k * 16, 16)]
            pv = qpos[pl.ds(k * 16, 16)]
            dv = qdl[pl.ds(k * 16, 16)]
            idxs[pl.ds(np * FB + k * 16, 16)] = sv
            idxp[pl.ds(np * FB + k * 16, 16)] = pv
            idxd2[np, pl.ds(k * 16, 16)] = dv
            idxr[pl.ds(np * FB + k * 16, 16)] = jnp.minimum(dv + lo, NT - 1)
        pltpu.async_copy(feat_hbm.at[idxs.at[pl.ds(np * FB, FB)]],
                         fbuf.at[pl.ds(np * FB, FB)], sem0)
        pltpu.async_copy(ee_hbm.at[idxp.at[pl.ds(np * FB, FB)]],
                         eebuf.at[pl.ds(np * FB, FB)], sem1)
        pltpu.async_copy(rd_hbm.at[idxr.at[pl.ds(np * FB, FB)]],
                         rdbuf.at[pl.ds(np * FB, FB)], sem2)
        for k in range((QN - FB) // 16):
            qsrc[pl.ds(k * 16, 16)] = qsrc[pl.ds(FB + k * 16, 16)]
            qdl[pl.ds(k * 16, 16)] = qdl[pl.ds(FB + k * 16, 16)]
            qpos[pl.ds(k * 16, 16)] = qpos[pl.ds(FB + k * 16, 16)]
        for k in range(FB // 16):
            qsrc[pl.ds(QN - FB + k * 16, 16)] = z16i
            qdl[pl.ds(QN - FB + k * 16, 16)] = dump16
            qpos[pl.ds(QN - FB + k * 16, 16)] = z16i

    def complete(ip, spin):
        @pl.when(spin == 1)
        def _():
            pltpu.make_async_copy(fbuf.at[pl.ds((1 - ip) * FB, FB)],
                                  rst_sh.at[idxd2.at[1 - ip]], sem3).wait()

        pltpu.make_async_copy(feat_hbm.at[idxs.at[pl.ds(ip * FB, FB)]],
                              fbuf.at[pl.ds(ip * FB, FB)], sem0).wait()
        pltpu.make_async_copy(ee_hbm.at[idxp.at[pl.ds(ip * FB, FB)]],
                              eebuf.at[pl.ds(ip * FB, FB)], sem1).wait()
        pltpu.make_async_copy(rd_hbm.at[idxr.at[pl.ds(ip * FB, FB)]],
                              rdbuf.at[pl.ds(ip * FB, FB)], sem2).wait()

        def mrow(i, carry):
            r = ip * FB + i
            iv = jnp.zeros((16,), _i32) + r
            for h in range(8):
                hv = jnp.full((16,), h, _i32)
                asp = (plsc.load_gather(eebuf, [iv, hv])
                       * plsc.load_gather(rdbuf, [iv, hv]))
                for cc in (2 * h, 2 * h + 1):
                    fbuf[r, pl.ds(cc * 16, 16)] = (
                        fbuf[r, pl.ds(cc * 16, 16)] * asp)
            return carry

        lax.fori_loop(0, FB, mrow, 0)
        pltpu.async_copy(fbuf.at[pl.ds(ip * FB, FB)],
                         rst_sh.at[idxd2.at[ip]], sem3, add=True)

    def maybe_flush(state, lo, thresh):
        q, pend, ip, sp = state
        hit = q >= thresh

        @pl.when(hit & (pend == 1))
        def _():
            complete(ip, sp)

        np = jnp.where(pend == 1, 1 - ip, 0)

        @pl.when(hit)
        def _():
            issue(np, lo)

        q = jnp.where(hit, q - FB, q)
        sp = jnp.where(hit & (pend == 1), 1, sp)
        pend = jnp.where(hit, 1, pend)
        ip = jnp.where(hit, np, ip)
        return (q, pend, ip, sp)

    def one_pass(p, carry):
        lo = (c * NPASS + p) * RNG
        lax.fori_loop(0, 32, zrow, 0)
        r0 = s * 289
        for t in range(9):
            pltpu.sync_copy(rbuf, rst_sh.at[pl.ds(r0 + t * 32, 32)])
        pltpu.sync_copy(rbuf.at[pl.ds(0, 1)], rst_sh.at[pl.ds(r0 + 288, 1)])
        queue_reset()
        plsc.subcore_barrier()

        hi = jnp.minimum(lo + RNG, NCOV)

        def process4(sb, db, row, state, lo):
            for j in range(4):
                q = state[0]
                for gg in range(8):
                    dv = db[j, pl.ds(gg * 16, 16)]
                    sv = sb[j, pl.ds(gg * 16, 16)]
                    mask = (dv >= lo) & (dv < hi)
                    m01 = jnp.where(mask, 1, 0).astype(_i32)
                    csum = plsc.cumsum(m01)
                    tgt = q + csum - 1
                    plsc.store_scatter(qsrc, [tgt], sv, mask=mask)
                    plsc.store_scatter(qdl, [tgt], dv - lo, mask=mask)
                    pos = (row + j) * 128 + gg * 16 + iota16
                    plsc.store_scatter(qpos, [tgt], pos, mask=mask)
                    q = q + jnp.sum(m01)
                state = (q,) + state[1:]
                state = maybe_flush(state, lo, FB)
            return state

        base = s * 392
        pltpu.async_copy(src_hbm.at[pl.ds(base, 4)], srcba, semas)
        pltpu.async_copy(dst_hbm.at[pl.ds(base, 4)], dstba, semad)
        pltpu.async_copy(src_hbm.at[pl.ds(base + 4, 4)], srcbb, sembs)
        pltpu.async_copy(dst_hbm.at[pl.ds(base + 4, 4)], dstbb, sembd)

        def chunk(g, state):
            rowa = base + g * 8
            pltpu.make_async_copy(src_hbm.at[pl.ds(rowa, 4)], srcba,
                                  semas).wait()
            pltpu.make_async_copy(dst_hbm.at[pl.ds(rowa, 4)], dstba,
                                  semad).wait()
            state = process4(srcba, dstba, rowa, state, lo)
            pltpu.async_copy(src_hbm.at[pl.ds(rowa + 8, 4)], srcba, semas)
            pltpu.async_copy(dst_hbm.at[pl.ds(rowa + 8, 4)], dstba, semad)
            rowb = rowa + 4
            pltpu.make_async_copy(src_hbm.at[pl.ds(rowb, 4)], srcbb,
                                  sembs).wait()
            pltpu.make_async_copy(dst_hbm.at[pl.ds(rowb, 4)], dstbb,
                                  sembd).wait()
            state = process4(srcbb, dstbb, rowb, state, lo)
            pltpu.async_copy(src_hbm.at[pl.ds(rowb + 8, 4)], srcbb, sembs)
            pltpu.async_copy(dst_hbm.at[pl.ds(rowb + 8, 4)], dstbb, sembd)
            return state

        state = lax.fori_loop(
            0, 49, chunk,
            (jnp.int32(0), jnp.int32(0), jnp.int32(0), jnp.int32(0)))
        pltpu.make_async_copy(src_hbm.at[pl.ds(base, 4)], srcba, semas).wait()
        pltpu.make_async_copy(dst_hbm.at[pl.ds(base, 4)], dstba, semad).wait()
        pltpu.make_async_copy(src_hbm.at[pl.ds(base, 4)], srcbb, sembs).wait()
        pltpu.make_async_copy(dst_hbm.at[pl.ds(base, 4)], dstbb, sembd).wait()
        q, pend, ip, sp = state
        f1 = pend == 1

        @pl.when(f1)
        def _():
            complete(ip, sp)

        sp1 = jnp.where(f1, 1, sp)
        np2 = jnp.where(f1, 1 - ip, 0)
        f2 = q >= 1

        @pl.when(f2)
        def _():
            issue(np2, lo)
            complete(np2, sp1)

        spf = jnp.where(f1 | f2, 1, sp)
        half = jnp.where(f2, np2, jnp.where(f1, ip, 1 - ip))

        @pl.when(spf == 1)
        def _():
            pltpu.make_async_copy(fbuf.at[pl.ds(half * FB, FB)],
                                  rst_sh.at[idxd2.at[half]], sem3).wait()

        plsc.subcore_barrier()

        r0p = s * 288
        for t in range(9):
            roff = r0p + t * 32
            pltpu.sync_copy(rst_sh.at[pl.ds(roff, 32)], rbuf)
            pltpu.sync_copy(fn_hbm.at[pl.ds(lo + roff, 32)], fidx32)

            def prow(i, carry2):
                for k in range(16):
                    v = rbuf[i, pl.ds(k * 16, 16)]
                    ev = jnp.exp(jnp.minimum(v, 0.0)) - 1.0
                    rbuf[i, pl.ds(k * 16, 16)] = jnp.where(v > 0.0, v, ev)
                return carry2

            lax.fori_loop(0, 32, prow, 0)
            pltpu.sync_copy(rbuf, pooled_sh.at[fidx32], add=True)
        plsc.subcore_barrier()
        return carry

    lax.fori_loop(0, NPASS, one_pass, 0)

    pltpu.sync_copy(pooled_sh.at[pl.ds(s * 4, 4)], pbuf)
    pltpu.sync_copy(pbuf, pp_hbm.at[c, pl.ds(s * 4, 4)])


def _b2(src2, dst2, feat, ee, rd, fn_pad):
    mesh = plsc.VectorSubcoreMesh(core_axis_name="c", subcore_axis_name="s",
                                  num_cores=NC, num_subcores=NS)
    f = pl.kernel(
        _b2_body,
        compiler_params=pltpu.CompilerParams(use_tc_tiling_on_sc=False, needs_layout_passes=False),
        out_type=jax.ShapeDtypeStruct((NC, G, HD), _f32),
        mesh=mesh,
        scratch_types=[
            pltpu.VMEM((4, 128), _i32),
            pltpu.VMEM((4, 128), _i32),
            pltpu.VMEM((4, 128), _i32),
            pltpu.VMEM((4, 128), _i32),
            pltpu.VMEM((QN,), _i32),
            pltpu.VMEM((QN,), _i32),
            pltpu.VMEM((QN,), _i32),
            pltpu.VMEM((2 * FB,), _i32),
            pltpu.VMEM((2 * FB,), _i32),
            pltpu.VMEM((2, FB), _i32),
            pltpu.VMEM((2 * FB,), _i32),
            pltpu.VMEM((2 * FB, HD), _f32),
            pltpu.VMEM((2 * FB, 16), _f32),
            pltpu.VMEM((2 * FB, 16), _f32),
            pltpu.VMEM((32, HD), _f32),
            pltpu.VMEM((32,), _i32),
            pltpu.VMEM((4, HD), _f32),
            pltpu.SemaphoreType.DMA,
            pltpu.SemaphoreType.DMA,
            pltpu.SemaphoreType.DMA,
            pltpu.SemaphoreType.DMA,
            pltpu.SemaphoreType.DMA,
            pltpu.SemaphoreType.DMA,
            pltpu.SemaphoreType.DMA,
            pltpu.SemaphoreType.DMA,
            pltpu.VMEM_SHARED((RNG + 16, HD), _f32),
            pltpu.VMEM_SHARED((G, HD), _f32),
        ],
    )
    return f(src2, dst2, feat, ee, rd, fn_pad)


def _tcc_body(fn_ref, pp_ref, bg_ref, wc_ref, bc_ref, out_ref, bge_ref,
              cnt_ref):
    i = pl.program_id(0)

    @pl.when(i == 0)
    def _():
        cnt_ref[...] = jnp.zeros((G, 128), _f32)

    ids = fn_ref[0]
    io = lax.broadcasted_iota(_i32, (2000, G), 1)
    oh = (ids == io).astype(_f32)
    ones = jnp.ones((2000, 1), _f32)
    cnt = lax.dot_general(oh, ones, (((0,), (0,)), ((), ())),
                          preferred_element_type=_f32)
    cnt_ref[:, 0:1] += cnt

    @pl.when(i == 24)
    def _():
        rc = 1.0 / jnp.maximum(cnt_ref[:, 0:1], 1.0)
        pooled = (pp_ref[0] + pp_ref[1]) * rc
        bio = lax.broadcasted_iota(_i32, (G, G), 1)
        ohg = (bg_ref[...] == bio).astype(_f32)
        bge = jnp.dot(ohg, pooled, preferred_element_type=_f32)
        bge_ref[...] = bge
        out_ref[...] = jnp.dot(bge, wc_ref[...],
                               preferred_element_type=_f32) + bc_ref[...]


def _tcc(fn_cols, pp, bg_col, wc, bc2):
    return pl.pallas_call(
        _tcc_body,
        grid=(25,),
        in_specs=[
            pl.BlockSpec((1, 2000, 1), lambda i: (i, 0, 0)),
            pl.BlockSpec((2, G, HD), lambda i: (0, 0, 0)),
            pl.BlockSpec((G, 1), lambda i: (0, 0)),
            pl.BlockSpec((HD, 2), lambda i: (0, 0)),
            pl.BlockSpec((1, 2), lambda i: (0, 0)),
        ],
        out_specs=[
            pl.BlockSpec((G, 2), lambda i: (0, 0)),
            pl.BlockSpec((G, HD), lambda i: (0, 0)),
        ],
        out_shape=[
            jax.ShapeDtypeStruct((G, 2), _f32),
            jax.ShapeDtypeStruct((G, HD), _f32),
        ],
        scratch_shapes=[pltpu.VMEM((G, 128), _f32)],
    )(fn_cols, pp, bg_col, wc, bc2)


@jax.jit
def kernel(x, edge_index, filename_ids, batched_g_ids, W, attn_l, attn_r,
           sa_W1, sa_b1, sa_W2, Wc, bc):
    x_pad = jnp.zeros((NP, 8), _f32).at[:N].set(x.astype(_f32))
    hrow = jnp.arange(HD, dtype=_i32) // D
    hcol = jnp.arange(H, dtype=_i32)
    sel = (hrow[:, None] == hcol[None, :]).astype(_f32)
    al = sel * attn_l.reshape(HD)[:, None]
    ar = sel * attn_r.reshape(HD)[:, None]

    src = edge_index[0].astype(_i32)
    dst = edge_index[1].astype(_i32)
    src2 = jnp.zeros((EP + 1024,), _i32).at[:E].set(src).reshape(ER + 8, 128)
    dst2 = jnp.full((EP + 1024,), PADDST, _i32).at[:E].set(dst).reshape(
        ER + 8, 128)

    fn_pad = jnp.zeros((55296,), _i32).at[:N].set(filename_ids.astype(_i32))
    fn_cols = filename_ids.astype(_i32).reshape(25, 2000, 1)
    bg_col = batched_g_ids.astype(_i32).reshape(G, 1)
    bc2 = bc.reshape(1, 2).astype(_f32)

    feat, el_t, er_t, cap = _tca(x_pad, W.astype(_f32), al, ar)
    ee, dp = _b1(src2, dst2, el_t, er_t, cap)
    rd = _a2(dp)
    pp = _b2(src2, dst2, feat, ee, rd, fn_pad)
    out, bge = _tcc(fn_cols, pp, bg_col, Wc.astype(_f32), bc2)
    return (out, bge)
